# Initial kernel scaffold; baseline (speedup 1.0000x reference)
import jax
import jax.numpy as jnp
from jax.experimental import pallas as pl


def kernel(x, edge_index, batch, W_rel1, b_rel1, W_root1, W_rels, b_rels, W_roots, W_lin1, b_lin1, W_lin2, b_lin2):
    raise NotImplementedError("write your pallas kernel here")


# speedup vs baseline: 10.1518x; 10.1518x over previous
import functools

import jax
import jax.numpy as jnp
from jax import lax
from jax.experimental import pallas as pl
from jax.experimental.pallas import tpu as pltpu
from jax.experimental.pallas import tpu_sc as plsc

N = 100000
E = 1600000
G = 512
H = 32
F_IN = 14

NC = 2
NS = 16
LANES = 16

BLK = 1024
NB = 98
NA = NB * BLK
ROWS_PER_SUB = NA // NS

EROWS = 12544
EP = EROWS * 128
ROWS_PER_SUB_E = EROWS // NS
ROWS_PER_WORKER_E = EROWS // (NC * NS)
CHUNK_ROWS = 8

@functools.cache
def _mesh():
    return plsc.VectorSubcoreMesh(core_axis_name="c", subcore_axis_name="s")


_SC_PARAMS = pltpu.CompilerParams(use_tc_tiling_on_sc=False)


def _sc_edge_body(table, src_hbm, dst_hbm, zeros_hbm, out_hbm,
                  srcv, dstv, rows, acc, gsem, ssem, *, row0, nrows):
    s = lax.axis_index("s")
    pltpu.sync_copy(zeros_hbm, acc.at[pl.ds(s * ROWS_PER_SUB, ROWS_PER_SUB)])
    plsc.subcore_barrier()

    nchunks = nrows // CHUNK_ROWS

    @pl.loop(0, nchunks)
    def _(k):
        base = row0 + k * CHUNK_ROWS
        pltpu.sync_copy(src_hbm.at[pl.ds(base, CHUNK_ROWS)], srcv)
        pltpu.sync_copy(dst_hbm.at[pl.ds(base, CHUNK_ROWS)], dstv)
        gets = [pltpu.async_copy(table.at[srcv.at[j]],
                                 rows.at[pl.ds(j * 128, 128)], gsem)
                for j in range(CHUNK_ROWS)]
        for g_ in gets:
            g_.wait()
        puts = [pltpu.async_copy(rows.at[pl.ds(j * 128, 128)],
                                 acc.at[dstv.at[j]], ssem, add=True)
                for j in range(CHUNK_ROWS)]
        for p_ in puts:
            p_.wait()

    plsc.subcore_barrier()
    pltpu.sync_copy(acc.at[pl.ds(s * ROWS_PER_SUB, ROWS_PER_SUB)],
                    out_hbm.at[pl.ds(s * ROWS_PER_SUB, ROWS_PER_SUB)])


def _sc_scratch():
    return [
        pltpu.VMEM((CHUNK_ROWS, 128), jnp.int32),
        pltpu.VMEM((CHUNK_ROWS, 128), jnp.int32),
        pltpu.VMEM((BLK, LANES), jnp.float32),
        pltpu.VMEM_SHARED((NA, LANES), jnp.float32),
        pltpu.SemaphoreType.DMA,
        pltpu.SemaphoreType.DMA,
    ]


@jax.jit
def _sc_layer1(x_pad, src2d, dst2d, zeros):
    @functools.partial(
        pl.kernel,
        out_type=jax.ShapeDtypeStruct((NC, NA, LANES), jnp.float32),
        mesh=_mesh(), scratch_types=_sc_scratch(),
        compiler_params=_SC_PARAMS)
    def k(x_hbm, src_hbm, dst_hbm, z_hbm, out_hbm,
          srcv, dstv, rows, acc, gsem, ssem):
        c = lax.axis_index("c")
        s = lax.axis_index("s")
        w = c * NS + s
        _sc_edge_body(x_hbm, src_hbm, dst_hbm, z_hbm, out_hbm.at[c],
                      srcv, dstv, rows, acc, gsem, ssem,
                      row0=w * ROWS_PER_WORKER_E, nrows=ROWS_PER_WORKER_E)
    return k(x_pad, src2d, dst2d, zeros)


@jax.jit
def _sc_layer(h, src2d, dst2d, zeros):
    @functools.partial(
        pl.kernel,
        out_type=jax.ShapeDtypeStruct((NC, NA, LANES), jnp.float32),
        mesh=_mesh(), scratch_types=_sc_scratch(),
        compiler_params=_SC_PARAMS)
    def k(h_hbm, src_hbm, dst_hbm, z_hbm, out_hbm,
          srcv, dstv, rows, acc, gsem, ssem):
        c = lax.axis_index("c")
        s = lax.axis_index("s")
        _sc_edge_body(h_hbm.at[c], src_hbm, dst_hbm, z_hbm, out_hbm.at[c],
                      srcv, dstv, rows, acc, gsem, ssem,
                      row0=s * ROWS_PER_SUB_E, nrows=ROWS_PER_SUB_E)
    return k(h, src2d, dst2d, zeros)


_mxu = functools.partial(
    lax.dot_general, dimension_numbers=(((1,), (0,)), ((), ())),
    preferred_element_type=jnp.float32)


def _dotbf(a, b):
    return _mxu(a.astype(jnp.bfloat16), b.astype(jnp.bfloat16))


def _dot3(a, b):
    ah = a.astype(jnp.bfloat16)
    al = (a - ah.astype(jnp.float32)).astype(jnp.bfloat16)
    bh = b.astype(jnp.bfloat16)
    bl = (b - bh.astype(jnp.float32)).astype(jnp.bfloat16)
    return _mxu(ah, bh) + (_mxu(ah, bl) + (_mxu(al, bh) + _mxu(al, bl)))


def _tc_layer1_body(p_ref, x_ref, wr_ref, br_ref, wo_ref, o_ref):
    agg = p_ref[0] + p_ref[1]
    o = (_dotbf(agg, wr_ref[...].T) + br_ref[...]
         + _dotbf(x_ref[...], wo_ref[...].T))
    o = jnp.maximum(o, 0.0)
    o_ref[0] = o[:, :LANES]
    o_ref[1] = o[:, LANES:]


@jax.jit
def _tc_layer1(part, x_pad, wr, br, wo):
    return pl.pallas_call(
        _tc_layer1_body,
        grid=(NB,),
        in_specs=[
            pl.BlockSpec((NC, BLK, LANES), lambda i: (0, i, 0)),
            pl.BlockSpec((BLK, LANES), lambda i: (i, 0)),
            pl.BlockSpec((H, LANES), lambda i: (0, 0)),
            pl.BlockSpec((1, H), lambda i: (0, 0)),
            pl.BlockSpec((H, LANES), lambda i: (0, 0)),
        ],
        out_specs=pl.BlockSpec((NC, BLK, LANES), lambda i: (0, i, 0)),
        out_shape=jax.ShapeDtypeStruct((NC, NA, LANES), jnp.float32),
    )(part, x_pad, wr, br, wo)


def _tc_layer_body(a_ref, h_ref, wr_ref, br_ref, wo_ref, o_ref):
    agg = jnp.concatenate([a_ref[0], a_ref[1]], axis=1)
    hh = jnp.concatenate([h_ref[0], h_ref[1]], axis=1)
    o = (_dotbf(agg, wr_ref[...].T) + br_ref[...]
         + _dotbf(hh, wo_ref[...].T))
    o = jnp.maximum(o, 0.0)
    o_ref[0] = o[:, :LANES]
    o_ref[1] = o[:, LANES:]


@jax.jit
def _tc_layer(agg, h, wr, br, wo):
    return pl.pallas_call(
        _tc_layer_body,
        grid=(NB,),
        in_specs=[
            pl.BlockSpec((NC, BLK, LANES), lambda i: (0, i, 0)),
            pl.BlockSpec((NC, BLK, LANES), lambda i: (0, i, 0)),
            pl.BlockSpec((H, H), lambda i: (0, 0)),
            pl.BlockSpec((1, H), lambda i: (0, 0)),
            pl.BlockSpec((H, H), lambda i: (0, 0)),
        ],
        out_specs=pl.BlockSpec((NC, BLK, LANES), lambda i: (0, i, 0)),
        out_shape=jax.ShapeDtypeStruct((NC, NA, LANES), jnp.float32),
    )(agg, h, wr, br, wo)


def _tc_pool_body(h_ref, b_ref, w1_ref, b1_ref, w2_ref, b2_ref, o_ref, acc):
    i = pl.program_id(0)

    @pl.when(i == 0)
    def _():
        acc[...] = jnp.zeros_like(acc)

    hh = jnp.concatenate([h_ref[0], h_ref[1]], axis=1)
    bat = b_ref[0, 0, :]
    gid = lax.broadcasted_iota(jnp.int32, (G, BLK), 0)
    onehot = (gid == bat[None, :]).astype(jnp.float32)
    acc[...] += _dot3(onehot, hh)

    @pl.when(i == NB - 1)
    def _():
        h1 = jnp.maximum(_dotbf(acc[...], w1_ref[...].T) + b1_ref[...], 0.0)
        logits = _dotbf(h1, w2_ref[...].T) + b2_ref[...]
        m = jnp.max(logits, axis=1, keepdims=True)
        lse = m + jnp.log(jnp.sum(jnp.exp(logits - m), axis=1, keepdims=True))
        o_ref[...] = logits - lse


@jax.jit
def _tc_pool(h, batch3d, w1, b1, w2, b2):
    return pl.pallas_call(
        _tc_pool_body,
        grid=(NB,),
        in_specs=[
            pl.BlockSpec((NC, BLK, LANES), lambda i: (0, i, 0)),
            pl.BlockSpec((1, 1, BLK), lambda i: (i, 0, 0)),
            pl.BlockSpec((H, H), lambda i: (0, 0)),
            pl.BlockSpec((1, H), lambda i: (0, 0)),
            pl.BlockSpec((2, H), lambda i: (0, 0)),
            pl.BlockSpec((1, 2), lambda i: (0, 0)),
        ],
        out_specs=pl.BlockSpec((G, 2), lambda i: (0, 0)),
        out_shape=jax.ShapeDtypeStruct((G, 2), jnp.float32),
        scratch_shapes=[pltpu.VMEM((G, H), jnp.float32)],
    )(h, batch3d, w1, b1, w2, b2)


def kernel(x, edge_index, batch, W_rel1, b_rel1, W_root1, W_rels, b_rels,
           W_roots, W_lin1, b_lin1, W_lin2, b_lin2):
    src2d = jnp.reshape(
        jnp.pad(edge_index[0], (0, EP - E)), (EROWS, 128))
    dst2d = jnp.reshape(
        jnp.pad(edge_index[1], (0, EP - E), constant_values=N), (EROWS, 128))
    x_pad = jnp.pad(x, ((0, NA - N), (0, LANES - F_IN)))
    zeros = jnp.zeros((ROWS_PER_SUB, LANES), jnp.float32)
    batch3d = jnp.reshape(
        jnp.pad(batch, (0, NA - N), constant_values=G), (NB, 1, BLK))
    wr1 = jnp.pad(W_rel1, ((0, 0), (0, LANES - F_IN)))
    wo1 = jnp.pad(W_root1, ((0, 0), (0, LANES - F_IN)))

    part = _sc_layer1(x_pad, src2d, dst2d, zeros)
    h = _tc_layer1(part, x_pad, wr1, b_rel1.reshape(1, H), wo1)

    for i in range(4):
        agg = _sc_layer(h, src2d, dst2d, zeros)
        h = _tc_layer(agg, h, W_rels[i], b_rels[i].reshape(1, H), W_roots[i])

    return _tc_pool(h, batch3d, W_lin1, b_lin1.reshape(1, H),
                    W_lin2, b_lin2.reshape(1, 2))


# speedup vs baseline: 15.8209x; 1.5584x over previous
import functools

import jax
import jax.numpy as jnp
from jax import lax
from jax.experimental import pallas as pl
from jax.experimental.pallas import tpu as pltpu
from jax.experimental.pallas import tpu_sc as plsc

N = 100000
E = 1600000
G = 512
H = 32
F_IN = 14

NC = 2
NS = 16
LANES = 16

BLK = 1024
NB = 98
NA = NB * BLK
ROWS_PER_SUB = NA // NS

R = NA // 8
RB = 1792
NRB = R // RB

EROWS = 12544
EP = EROWS * 128
ROWS_PER_SUB_E = EROWS // NS
ROWS_PER_WORKER_E = EROWS // (NC * NS)
CHUNK_ROWS = 8

@functools.cache
def _mesh():
    return plsc.VectorSubcoreMesh(core_axis_name="c", subcore_axis_name="s")


_SC_PARAMS = pltpu.CompilerParams(use_tc_tiling_on_sc=False)


def _sc_edge_body(table, src_hbm, dst_hbm, zeros_hbm, out_hbm,
                  srcv, dstv, rows, acc, gsem, ssem, *, row0, nrows):
    s = lax.axis_index("s")
    pltpu.sync_copy(zeros_hbm, acc.at[pl.ds(s * ROWS_PER_SUB, ROWS_PER_SUB)])
    plsc.subcore_barrier()

    nchunks = nrows // CHUNK_ROWS

    @pl.loop(0, nchunks)
    def _(k):
        base = row0 + k * CHUNK_ROWS
        pltpu.sync_copy(src_hbm.at[pl.ds(base, CHUNK_ROWS)], srcv)
        pltpu.sync_copy(dst_hbm.at[pl.ds(base, CHUNK_ROWS)], dstv)
        gets = [pltpu.async_copy(table.at[srcv.at[j]],
                                 rows.at[pl.ds(j * 128, 128)], gsem)
                for j in range(CHUNK_ROWS)]
        for g_ in gets:
            g_.wait()
        puts = [pltpu.async_copy(rows.at[pl.ds(j * 128, 128)],
                                 acc.at[dstv.at[j]], ssem, add=True)
                for j in range(CHUNK_ROWS)]
        for p_ in puts:
            p_.wait()

    plsc.subcore_barrier()
    pltpu.sync_copy(acc.at[pl.ds(s * ROWS_PER_SUB, ROWS_PER_SUB)],
                    out_hbm.at[pl.ds(s * ROWS_PER_SUB, ROWS_PER_SUB)])


def _sc_scratch():
    return [
        pltpu.VMEM((CHUNK_ROWS, 128), jnp.int32),
        pltpu.VMEM((CHUNK_ROWS, 128), jnp.int32),
        pltpu.VMEM((BLK, LANES), jnp.float32),
        pltpu.VMEM_SHARED((NA, LANES), jnp.float32),
        pltpu.SemaphoreType.DMA,
        pltpu.SemaphoreType.DMA,
    ]


@jax.jit
def _sc_layer1(x_pad, src2d, dst2d, zeros):
    @functools.partial(
        pl.kernel,
        out_type=jax.ShapeDtypeStruct((NC, NA, LANES), jnp.float32),
        mesh=_mesh(), scratch_types=_sc_scratch(),
        compiler_params=_SC_PARAMS)
    def k(x_hbm, src_hbm, dst_hbm, z_hbm, out_hbm,
          srcv, dstv, rows, acc, gsem, ssem):
        c = lax.axis_index("c")
        s = lax.axis_index("s")
        w = c * NS + s
        _sc_edge_body(x_hbm, src_hbm, dst_hbm, z_hbm, out_hbm.at[c],
                      srcv, dstv, rows, acc, gsem, ssem,
                      row0=w * ROWS_PER_WORKER_E, nrows=ROWS_PER_WORKER_E)
    return k(x_pad, src2d, dst2d, zeros)


@jax.jit
def _sc_layer(h, src2d, dst2d, zeros):
    @functools.partial(
        pl.kernel,
        out_type=jax.ShapeDtypeStruct((NC, NA, LANES), jnp.float32),
        mesh=_mesh(), scratch_types=_sc_scratch(),
        compiler_params=_SC_PARAMS)
    def k(h_hbm, src_hbm, dst_hbm, z_hbm, out_hbm,
          srcv, dstv, rows, acc, gsem, ssem):
        c = lax.axis_index("c")
        s = lax.axis_index("s")
        _sc_edge_body(h_hbm.at[c], src_hbm, dst_hbm, z_hbm, out_hbm.at[c],
                      srcv, dstv, rows, acc, gsem, ssem,
                      row0=s * ROWS_PER_SUB_E, nrows=ROWS_PER_SUB_E)
    return k(h, src2d, dst2d, zeros)


_mxu = functools.partial(
    lax.dot_general, dimension_numbers=(((1,), (0,)), ((), ())),
    preferred_element_type=jnp.float32)


def _dotbf(a, b):
    return _mxu(a.astype(jnp.bfloat16), b.astype(jnp.bfloat16))


def _dot3(a, b):
    ah = a.astype(jnp.bfloat16)
    al = (a - ah.astype(jnp.float32)).astype(jnp.bfloat16)
    bh = b.astype(jnp.bfloat16)
    bl = (b - bh.astype(jnp.float32)).astype(jnp.bfloat16)
    return _mxu(ah, bh) + (_mxu(ah, bl) + (_mxu(al, bh) + _mxu(al, bl)))


def _tc_layer1_body(p_ref, x_ref, br_r, bb_r, bo_r, o_ref):
    a = p_ref[0] + p_ref[1]
    xx = x_ref[...]
    for d in range(2):
        od = (_dotbf(a, br_r[d]) + bb_r[d]) + _dotbf(xx, bo_r[d])
        o_ref[d] = jnp.maximum(od, 0.0)


def _tc_layer1(part_p, x_p, br, bb, bo):
    return pl.pallas_call(
        _tc_layer1_body,
        grid=(NRB,),
        in_specs=[
            pl.BlockSpec((NC, RB, 128), lambda i: (0, i, 0)),
            pl.BlockSpec((RB, 128), lambda i: (i, 0)),
            pl.BlockSpec((2, 128, 128), lambda i: (0, 0, 0)),
            pl.BlockSpec((2, 128), lambda i: (0, 0)),
            pl.BlockSpec((2, 128, 128), lambda i: (0, 0, 0)),
        ],
        out_specs=pl.BlockSpec((NC, RB, 128), lambda i: (0, i, 0)),
        out_shape=jax.ShapeDtypeStruct((NC, R, 128), jnp.float32),
    )(part_p, x_p, br, bb, bo)


def _packed_dense(a_ref, h_ref, br_r, bb_r, bo_r):
    out = []
    for d in range(2):
        ma = _dotbf(a_ref[0], br_r[0, d]) + _dotbf(a_ref[1], br_r[1, d])
        mh = _dotbf(h_ref[0], bo_r[0, d]) + _dotbf(h_ref[1], bo_r[1, d])
        out.append(jnp.maximum((ma + bb_r[d]) + mh, 0.0))
    return out


def _tc_layer_body(a_ref, h_ref, br_r, bb_r, bo_r, o_ref):
    o = _packed_dense(a_ref, h_ref, br_r, bb_r, bo_r)
    o_ref[0] = o[0]
    o_ref[1] = o[1]


def _tc_layer(agg_p, h_p, br, bb, bo):
    return pl.pallas_call(
        _tc_layer_body,
        grid=(NRB,),
        in_specs=[
            pl.BlockSpec((NC, RB, 128), lambda i: (0, i, 0)),
            pl.BlockSpec((NC, RB, 128), lambda i: (0, i, 0)),
            pl.BlockSpec((2, 2, 128, 128), lambda i: (0, 0, 0, 0)),
            pl.BlockSpec((2, 128), lambda i: (0, 0)),
            pl.BlockSpec((2, 2, 128, 128), lambda i: (0, 0, 0, 0)),
        ],
        out_specs=pl.BlockSpec((NC, RB, 128), lambda i: (0, i, 0)),
        out_shape=jax.ShapeDtypeStruct((NC, R, 128), jnp.float32),
    )(agg_p, h_p, br, bb, bo)


def _tc_tail_body(a_ref, h_ref, br_r, bb_r, bo_r, b8_ref,
                  w1_ref, b1_ref, w2_ref, b2_ref, o_ref, acc):
    i = pl.program_id(0)

    @pl.when(i == 0)
    def _():
        acc[...] = jnp.zeros_like(acc)

    o = _packed_dense(a_ref, h_ref, br_r, bb_r, bo_r)

    for j in range(8):
        bat = b8_ref[j]
        oh = (lax.broadcasted_iota(jnp.int32, (G, RB), 0)
              == bat[None, :]).astype(jnp.bfloat16)
        cols = []
        for c in range(2):
            s = o[c][:, 16 * j:16 * (j + 1)]
            hi = s.astype(jnp.bfloat16)
            r1 = s - hi.astype(jnp.float32)
            lo = r1.astype(jnp.bfloat16)
            lo2 = (r1 - lo.astype(jnp.float32)).astype(jnp.bfloat16)
            cols += [hi, lo, lo2]
        res = _mxu(oh, jnp.concatenate(cols, axis=1))
        acc[...] += jnp.concatenate(
            [(res[:, 0:16] + res[:, 16:32]) + res[:, 32:48],
             (res[:, 48:64] + res[:, 64:80]) + res[:, 80:96]],
            axis=1)

    @pl.when(i == NRB - 1)
    def _():
        h1 = jnp.maximum(_dotbf(acc[...], w1_ref[...].T) + b1_ref[...], 0.0)
        logits = _dotbf(h1, w2_ref[...].T) + b2_ref[...]
        m = jnp.max(logits, axis=1, keepdims=True)
        lse = m + jnp.log(jnp.sum(jnp.exp(logits - m), axis=1, keepdims=True))
        o_ref[...] = logits - lse


def _tc_tail(agg_p, h_p, br, bb, bo, batch8, w1, b1, w2, b2):
    return pl.pallas_call(
        _tc_tail_body,
        grid=(NRB,),
        in_specs=[
            pl.BlockSpec((NC, RB, 128), lambda i: (0, i, 0)),
            pl.BlockSpec((NC, RB, 128), lambda i: (0, i, 0)),
            pl.BlockSpec((2, 2, 128, 128), lambda i: (0, 0, 0, 0)),
            pl.BlockSpec((2, 128), lambda i: (0, 0)),
            pl.BlockSpec((2, 2, 128, 128), lambda i: (0, 0, 0, 0)),
            pl.BlockSpec((8, RB), lambda i: (0, i)),
            pl.BlockSpec((H, H), lambda i: (0, 0)),
            pl.BlockSpec((1, H), lambda i: (0, 0)),
            pl.BlockSpec((2, H), lambda i: (0, 0)),
            pl.BlockSpec((1, 2), lambda i: (0, 0)),
        ],
        out_specs=pl.BlockSpec((G, 2), lambda i: (0, 0)),
        out_shape=jax.ShapeDtypeStruct((G, 2), jnp.float32),
        scratch_shapes=[pltpu.VMEM((G, H), jnp.float32)],
    )(agg_p, h_p, br, bb, bo, batch8, w1, b1, w2, b2)


def _blockdiag(W, cin):
    eye8 = jnp.eye(8, dtype=W.dtype)
    return jnp.stack([
        jnp.stack([
            jnp.kron(eye8, W[16 * d:16 * (d + 1), 16 * c:16 * (c + 1)].T)
            for d in range(2)])
        for c in range(cin)])


def _packbias(b):
    return jnp.tile(b.reshape(2, 1, LANES), (1, 8, 1)).reshape(2, 128)


def kernel(x, edge_index, batch, W_rel1, b_rel1, W_root1, W_rels, b_rels,
           W_roots, W_lin1, b_lin1, W_lin2, b_lin2):
    src2d = jnp.reshape(
        jnp.pad(edge_index[0], (0, EP - E)), (EROWS, 128))
    dst2d = jnp.reshape(
        jnp.pad(edge_index[1], (0, EP - E), constant_values=N), (EROWS, 128))
    x_pad = jnp.pad(x, ((0, NA - N), (0, LANES - F_IN)))
    x_p = jnp.reshape(x_pad, (R, 128))
    zeros = jnp.zeros((ROWS_PER_SUB, LANES), jnp.float32)
    batch8 = jnp.reshape(
        jnp.pad(batch, (0, NA - N), constant_values=G), (R, 8)).T
    wr1 = jnp.pad(W_rel1, ((0, 0), (0, LANES - F_IN)))
    wo1 = jnp.pad(W_root1, ((0, 0), (0, LANES - F_IN)))

    part = _sc_layer1(x_pad, src2d, dst2d, zeros)
    h = _tc_layer1(jnp.reshape(part, (NC, R, 128)), x_p,
                   _blockdiag(wr1, 1)[0], _packbias(b_rel1),
                   _blockdiag(wo1, 1)[0])

    for i in range(3):
        agg = _sc_layer(jnp.reshape(h, (NC, NA, LANES)), src2d, dst2d, zeros)
        h = _tc_layer(jnp.reshape(agg, (NC, R, 128)), h,
                      _blockdiag(W_rels[i], 2), _packbias(b_rels[i]),
                      _blockdiag(W_roots[i], 2))

    agg = _sc_layer(jnp.reshape(h, (NC, NA, LANES)), src2d, dst2d, zeros)
    return _tc_tail(jnp.reshape(agg, (NC, R, 128)), h,
                    _blockdiag(W_rels[3], 2), _packbias(b_rels[3]),
                    _blockdiag(W_roots[3], 2), batch8,
                    W_lin1, b_lin1.reshape(1, H),
                    W_lin2, b_lin2.reshape(1, 2))


# speedup vs baseline: 19.9135x; 1.2587x over previous
import functools

import jax
import jax.numpy as jnp
from jax import lax
from jax.experimental import pallas as pl
from jax.experimental.pallas import tpu as pltpu
from jax.experimental.pallas import tpu_sc as plsc

N = 100000
E = 1600000
G = 512
H = 32
F_IN = 14

NC = 2
NS = 16
LANES = 16

BLK = 1024
NB = 98
NA = NB * BLK
ROWS_PER_SUB = NA // NS

R = NA // 8
RB = 1792
NRB = R // RB

EROWS = 12544
EP = EROWS * 128
ROWS_PER_SUB_E = EROWS // NS
ROWS_PER_WORKER_E = EROWS // (NC * NS)
CHUNK_ROWS = 4
CHUNK_EDGES = CHUNK_ROWS * 128

@functools.cache
def _mesh():
    return plsc.VectorSubcoreMesh(core_axis_name="c", subcore_axis_name="s")


_SC_PARAMS = pltpu.CompilerParams(use_tc_tiling_on_sc=False)


def _sc_edge_body(table, src_hbm, dst_hbm, zeros_hbm, out_hbm,
                  srcv, dstv, rows, acc, isem, gsem, ssem, *, row0, nrows):
    s = lax.axis_index("s")
    pltpu.sync_copy(zeros_hbm, acc.at[pl.ds(s * ROWS_PER_SUB, ROWS_PER_SUB)])
    plsc.subcore_barrier()

    nchunks = nrows // CHUNK_ROWS
    pltpu.async_copy(src_hbm.at[pl.ds(row0, CHUNK_ROWS)], srcv.at[0], isem)
    pltpu.async_copy(dst_hbm.at[pl.ds(row0, CHUNK_ROWS)], dstv.at[0], isem)

    @pl.loop(0, nchunks)
    def _(k):
        bb = lax.rem(k, 2)

        @pl.when(k >= 2)
        def _():
            pltpu.make_async_copy(zeros_hbm.at[pl.ds(0, CHUNK_EDGES)],
                                  rows.at[bb], ssem).wait()

        pltpu.make_async_copy(src_hbm.at[pl.ds(0, CHUNK_ROWS)],
                              srcv.at[bb], isem).wait()
        pltpu.make_async_copy(dst_hbm.at[pl.ds(0, CHUNK_ROWS)],
                              dstv.at[bb], isem).wait()

        @pl.when(k + 1 < nchunks)
        def _():
            nb = 1 - bb
            base = row0 + (k + 1) * CHUNK_ROWS
            pltpu.async_copy(src_hbm.at[pl.ds(base, CHUNK_ROWS)],
                             srcv.at[nb], isem)
            pltpu.async_copy(dst_hbm.at[pl.ds(base, CHUNK_ROWS)],
                             dstv.at[nb], isem)

        gets = [pltpu.async_copy(table.at[srcv.at[bb].at[j]],
                                 rows.at[bb].at[pl.ds(j * 128, 128)], gsem)
                for j in range(CHUNK_ROWS)]
        for g_ in gets:
            g_.wait()
        for j in range(CHUNK_ROWS):
            pltpu.async_copy(rows.at[bb].at[pl.ds(j * 128, 128)],
                             acc.at[dstv.at[bb].at[j]], ssem, add=True)

    pltpu.make_async_copy(zeros_hbm.at[pl.ds(0, CHUNK_EDGES)],
                          rows.at[0], ssem).wait()
    if nrows // CHUNK_ROWS >= 2:
        pltpu.make_async_copy(zeros_hbm.at[pl.ds(0, CHUNK_EDGES)],
                              rows.at[1], ssem).wait()

    plsc.subcore_barrier()
    pltpu.sync_copy(acc.at[pl.ds(s * ROWS_PER_SUB, ROWS_PER_SUB)],
                    out_hbm.at[pl.ds(s * ROWS_PER_SUB, ROWS_PER_SUB)])


def _sc_scratch():
    return [
        pltpu.VMEM((2, CHUNK_ROWS, 128), jnp.int32),
        pltpu.VMEM((2, CHUNK_ROWS, 128), jnp.int32),
        pltpu.VMEM((2, CHUNK_EDGES, LANES), jnp.float32),
        pltpu.VMEM_SHARED((NA, LANES), jnp.float32),
        pltpu.SemaphoreType.DMA,
        pltpu.SemaphoreType.DMA,
        pltpu.SemaphoreType.DMA,
    ]


@jax.jit
def _sc_layer1(x_pad, src2d, dst2d, zeros):
    @functools.partial(
        pl.kernel,
        out_type=jax.ShapeDtypeStruct((NC, NA, LANES), jnp.float32),
        mesh=_mesh(), scratch_types=_sc_scratch(),
        compiler_params=_SC_PARAMS)
    def k(x_hbm, src_hbm, dst_hbm, z_hbm, out_hbm,
          srcv, dstv, rows, acc, isem, gsem, ssem):
        c = lax.axis_index("c")
        s = lax.axis_index("s")
        w = c * NS + s
        _sc_edge_body(x_hbm, src_hbm, dst_hbm, z_hbm, out_hbm.at[c],
                      srcv, dstv, rows, acc, isem, gsem, ssem,
                      row0=w * ROWS_PER_WORKER_E, nrows=ROWS_PER_WORKER_E)
    return k(x_pad, src2d, dst2d, zeros)


@jax.jit
def _sc_layer(h, src2d, dst2d, zeros):
    @functools.partial(
        pl.kernel,
        out_type=jax.ShapeDtypeStruct((NC, NA, LANES), jnp.float32),
        mesh=_mesh(), scratch_types=_sc_scratch(),
        compiler_params=_SC_PARAMS)
    def k(h_hbm, src_hbm, dst_hbm, z_hbm, out_hbm,
          srcv, dstv, rows, acc, isem, gsem, ssem):
        c = lax.axis_index("c")
        s = lax.axis_index("s")
        _sc_edge_body(h_hbm.at[c], src_hbm, dst_hbm, z_hbm, out_hbm.at[c],
                      srcv, dstv, rows, acc, isem, gsem, ssem,
                      row0=s * ROWS_PER_SUB_E, nrows=ROWS_PER_SUB_E)
    return k(h, src2d, dst2d, zeros)


_mxu = functools.partial(
    lax.dot_general, dimension_numbers=(((1,), (0,)), ((), ())),
    preferred_element_type=jnp.float32)


def _dotbf(a, b):
    return _mxu(a.astype(jnp.bfloat16), b.astype(jnp.bfloat16))


def _dot3(a, b):
    ah = a.astype(jnp.bfloat16)
    al = (a - ah.astype(jnp.float32)).astype(jnp.bfloat16)
    bh = b.astype(jnp.bfloat16)
    bl = (b - bh.astype(jnp.float32)).astype(jnp.bfloat16)
    return _mxu(ah, bh) + (_mxu(ah, bl) + (_mxu(al, bh) + _mxu(al, bl)))


def _tc_layer1_body(p_ref, x_ref, br_r, bb_r, bo_r, o_ref):
    a = p_ref[0] + p_ref[1]
    xx = x_ref[...]
    for d in range(2):
        od = (_dotbf(a, br_r[d]) + bb_r[d]) + _dotbf(xx, bo_r[d])
        o_ref[d] = jnp.maximum(od, 0.0)


def _tc_layer1(part_p, x_p, br, bb, bo):
    return pl.pallas_call(
        _tc_layer1_body,
        grid=(NRB,),
        in_specs=[
            pl.BlockSpec((NC, RB, 128), lambda i: (0, i, 0)),
            pl.BlockSpec((RB, 128), lambda i: (i, 0)),
            pl.BlockSpec((2, 128, 128), lambda i: (0, 0, 0)),
            pl.BlockSpec((2, 128), lambda i: (0, 0)),
            pl.BlockSpec((2, 128, 128), lambda i: (0, 0, 0)),
        ],
        out_specs=pl.BlockSpec((NC, RB, 128), lambda i: (0, i, 0)),
        out_shape=jax.ShapeDtypeStruct((NC, R, 128), jnp.float32),
    )(part_p, x_p, br, bb, bo)


def _packed_dense(a_ref, h_ref, br_r, bb_r, bo_r):
    out = []
    for d in range(2):
        ma = _dotbf(a_ref[0], br_r[0, d]) + _dotbf(a_ref[1], br_r[1, d])
        mh = _dotbf(h_ref[0], bo_r[0, d]) + _dotbf(h_ref[1], bo_r[1, d])
        out.append(jnp.maximum((ma + bb_r[d]) + mh, 0.0))
    return out


def _tc_layer_body(a_ref, h_ref, br_r, bb_r, bo_r, o_ref):
    o = _packed_dense(a_ref, h_ref, br_r, bb_r, bo_r)
    o_ref[0] = o[0]
    o_ref[1] = o[1]


def _tc_layer(agg_p, h_p, br, bb, bo):
    return pl.pallas_call(
        _tc_layer_body,
        grid=(NRB,),
        in_specs=[
            pl.BlockSpec((NC, RB, 128), lambda i: (0, i, 0)),
            pl.BlockSpec((NC, RB, 128), lambda i: (0, i, 0)),
            pl.BlockSpec((2, 2, 128, 128), lambda i: (0, 0, 0, 0)),
            pl.BlockSpec((2, 128), lambda i: (0, 0)),
            pl.BlockSpec((2, 2, 128, 128), lambda i: (0, 0, 0, 0)),
        ],
        out_specs=pl.BlockSpec((NC, RB, 128), lambda i: (0, i, 0)),
        out_shape=jax.ShapeDtypeStruct((NC, R, 128), jnp.float32),
    )(agg_p, h_p, br, bb, bo)


def _tc_tail_body(a_ref, h_ref, br_r, bb_r, bo_r, b8_ref,
                  w1_ref, b1_ref, w2_ref, b2_ref, o_ref, acc):
    i = pl.program_id(0)

    @pl.when(i == 0)
    def _():
        acc[...] = jnp.zeros_like(acc)

    o = _packed_dense(a_ref, h_ref, br_r, bb_r, bo_r)

    for j in range(8):
        bat = b8_ref[j]
        oh = (lax.broadcasted_iota(jnp.int32, (G, RB), 0)
              == bat[None, :]).astype(jnp.bfloat16)
        cols = []
        for c in range(2):
            s = o[c][:, 16 * j:16 * (j + 1)]
            hi = s.astype(jnp.bfloat16)
            r1 = s - hi.astype(jnp.float32)
            lo = r1.astype(jnp.bfloat16)
            lo2 = (r1 - lo.astype(jnp.float32)).astype(jnp.bfloat16)
            cols += [hi, lo, lo2]
        res = _mxu(oh, jnp.concatenate(cols, axis=1))
        acc[...] += jnp.concatenate(
            [(res[:, 0:16] + res[:, 16:32]) + res[:, 32:48],
             (res[:, 48:64] + res[:, 64:80]) + res[:, 80:96]],
            axis=1)

    @pl.when(i == NRB - 1)
    def _():
        h1 = jnp.maximum(_dotbf(acc[...], w1_ref[...].T) + b1_ref[...], 0.0)
        logits = _dotbf(h1, w2_ref[...].T) + b2_ref[...]
        m = jnp.max(logits, axis=1, keepdims=True)
        lse = m + jnp.log(jnp.sum(jnp.exp(logits - m), axis=1, keepdims=True))
        o_ref[...] = logits - lse


def _tc_tail(agg_p, h_p, br, bb, bo, batch8, w1, b1, w2, b2):
    return pl.pallas_call(
        _tc_tail_body,
        grid=(NRB,),
        in_specs=[
            pl.BlockSpec((NC, RB, 128), lambda i: (0, i, 0)),
            pl.BlockSpec((NC, RB, 128), lambda i: (0, i, 0)),
            pl.BlockSpec((2, 2, 128, 128), lambda i: (0, 0, 0, 0)),
            pl.BlockSpec((2, 128), lambda i: (0, 0)),
            pl.BlockSpec((2, 2, 128, 128), lambda i: (0, 0, 0, 0)),
            pl.BlockSpec((8, RB), lambda i: (0, i)),
            pl.BlockSpec((H, H), lambda i: (0, 0)),
            pl.BlockSpec((1, H), lambda i: (0, 0)),
            pl.BlockSpec((2, H), lambda i: (0, 0)),
            pl.BlockSpec((1, 2), lambda i: (0, 0)),
        ],
        out_specs=pl.BlockSpec((G, 2), lambda i: (0, 0)),
        out_shape=jax.ShapeDtypeStruct((G, 2), jnp.float32),
        scratch_shapes=[pltpu.VMEM((G, H), jnp.float32)],
    )(agg_p, h_p, br, bb, bo, batch8, w1, b1, w2, b2)


def _blockdiag(W, cin):
    eye8 = jnp.eye(8, dtype=W.dtype)
    return jnp.stack([
        jnp.stack([
            jnp.kron(eye8, W[16 * d:16 * (d + 1), 16 * c:16 * (c + 1)].T)
            for d in range(2)])
        for c in range(cin)])


def _packbias(b):
    return jnp.tile(b.reshape(2, 1, LANES), (1, 8, 1)).reshape(2, 128)


def kernel(x, edge_index, batch, W_rel1, b_rel1, W_root1, W_rels, b_rels,
           W_roots, W_lin1, b_lin1, W_lin2, b_lin2):
    src2d = jnp.reshape(
        jnp.pad(edge_index[0], (0, EP - E)), (EROWS, 128))
    dst2d = jnp.reshape(
        jnp.pad(edge_index[1], (0, EP - E), constant_values=N), (EROWS, 128))
    x_pad = jnp.pad(x, ((0, NA - N), (0, LANES - F_IN)))
    x_p = jnp.reshape(x_pad, (R, 128))
    zeros = jnp.zeros((ROWS_PER_SUB, LANES), jnp.float32)
    batch8 = jnp.reshape(
        jnp.pad(batch, (0, NA - N), constant_values=G), (R, 8)).T
    wr1 = jnp.pad(W_rel1, ((0, 0), (0, LANES - F_IN)))
    wo1 = jnp.pad(W_root1, ((0, 0), (0, LANES - F_IN)))

    part = _sc_layer1(x_pad, src2d, dst2d, zeros)
    h = _tc_layer1(jnp.reshape(part, (NC, R, 128)), x_p,
                   _blockdiag(wr1, 1)[0], _packbias(b_rel1),
                   _blockdiag(wo1, 1)[0])

    for i in range(3):
        agg = _sc_layer(jnp.reshape(h, (NC, NA, LANES)), src2d, dst2d, zeros)
        h = _tc_layer(jnp.reshape(agg, (NC, R, 128)), h,
                      _blockdiag(W_rels[i], 2), _packbias(b_rels[i]),
                      _blockdiag(W_roots[i], 2))

    agg = _sc_layer(jnp.reshape(h, (NC, NA, LANES)), src2d, dst2d, zeros)
    return _tc_tail(jnp.reshape(agg, (NC, R, 128)), h,
                    _blockdiag(W_rels[3], 2), _packbias(b_rels[3]),
                    _blockdiag(W_roots[3], 2), batch8,
                    W_lin1, b_lin1.reshape(1, H),
                    W_lin2, b_lin2.reshape(1, 2))
